# baseline (speedup 1.0000x reference)
import functools

import jax
import jax.numpy as jnp
from jax import lax
from jax.experimental import pallas as pl
from jax.experimental.pallas import tpu as pltpu
from jax.experimental.pallas import tpu_sc as plsc

D_MODEL = 1024
MAX_LEN = 8192
BATCH = 2
SEQ = 8192

NUM_CORES = 2
NUM_SUBCORES = 16
NW = NUM_CORES * NUM_SUBCORES
S_PER_W = SEQ // NW
K = 8
N_CHUNKS = S_PER_W // K
DEPTH = 4
PREF = 3
N_ROUNDS = N_CHUNKS // DEPTH
LANES = 16
VPR = D_MODEL // LANES

_mesh = plsc.VectorSubcoreMesh(core_axis_name="c", subcore_axis_name="s")


@functools.partial(
    pl.kernel,
    out_type=jax.ShapeDtypeStruct((BATCH, SEQ, D_MODEL), jnp.float32),
    mesh=_mesh,
    scratch_types=[
        pltpu.VMEM((S_PER_W,), jnp.int32),
        pltpu.VMEM((DEPTH, K, D_MODEL), jnp.float32),
        pltpu.VMEM((DEPTH, BATCH, K, D_MODEL), jnp.float32),
        pltpu.SemaphoreType.DMA((DEPTH,)),
        pltpu.SemaphoreType.DMA((DEPTH,)),
        pltpu.SemaphoreType.DMA((DEPTH,)),
    ],
)
def _pe_add(inp_hbm, idx_hbm, pe_hbm, out_hbm, idx_v, pe_buf, in_buf,
            sem_pe, sem_in, sem_out):
    wid = lax.axis_index("s") * NUM_CORES + lax.axis_index("c")
    base = wid * S_PER_W

    def issue_pe(c, slot):
        off = pl.multiple_of(c * K, 8)
        pltpu.async_copy(pe_hbm.at[idx_v.at[pl.ds(off, K)]],
                         pe_buf.at[slot], sem_pe.at[slot])

    def issue_in(c, slot):
        s0 = pl.multiple_of(base + c * K, 8)
        pltpu.async_copy(inp_hbm.at[:, pl.ds(s0, K)],
                         in_buf.at[slot], sem_in.at[slot])

    def issue_loads(c, slot):
        issue_pe(c, slot)
        issue_in(c, slot)

    def wait_loads(slot):
        pltpu.make_async_copy(pe_hbm.at[pl.ds(0, K)], pe_buf.at[slot],
                              sem_pe.at[slot]).wait()
        pltpu.make_async_copy(inp_hbm.at[:, pl.ds(0, K)],
                              in_buf.at[slot], sem_in.at[slot]).wait()

    def issue_stores(c, slot):
        s0 = pl.multiple_of(base + c * K, 8)
        pltpu.async_copy(in_buf.at[slot], out_hbm.at[:, pl.ds(s0, K)],
                         sem_out.at[slot])

    def wait_stores(slot):
        pltpu.make_async_copy(in_buf.at[slot],
                              out_hbm.at[:, pl.ds(0, K)],
                              sem_out.at[slot]).wait()

    for c in range(PREF):
        issue_in(c, c % DEPTH)
    pltpu.sync_copy(idx_hbm.at[pl.ds(pl.multiple_of(base, 8), S_PER_W)],
                    idx_v)
    for c in range(PREF):
        issue_pe(c, c % DEPTH)

    def round_body(g, _):
        c0 = g * DEPTH
        for d in range(DEPTH):
            c = c0 + d
            nslot = (d + PREF) % DEPTH

            @pl.when(c + PREF < N_CHUNKS)
            def _(c=c, nslot=nslot):
                issue_pe(c + PREF, nslot)

                @pl.when(c >= DEPTH - PREF)
                def _():
                    wait_stores(nslot)
                issue_in(c + PREF, nslot)

            wait_loads(d)

            @plsc.parallel_loop(0, K)
            def _(r, d=d):
                for v in range(VPR):
                    sl = pl.ds(v * LANES, LANES)
                    pv = pe_buf[d, r, sl]
                    plsc.addupdate(in_buf.at[d, 0, r, sl], pv)
                    plsc.addupdate(in_buf.at[d, 1, r, sl], pv)

            issue_stores(c, d)
        return 0

    lax.fori_loop(0, N_ROUNDS, round_body, 0)

    for slot in range(DEPTH):
        wait_stores(slot)


def kernel(input, indices, pe):
    return _pe_add(input, indices.astype(jnp.int32), pe)


# speedup vs baseline: 1.0044x; 1.0044x over previous
import functools

import jax
import jax.numpy as jnp
from jax import lax
from jax.experimental import pallas as pl
from jax.experimental.pallas import tpu as pltpu
from jax.experimental.pallas import tpu_sc as plsc

D_MODEL = 1024
MAX_LEN = 8192
BATCH = 2
SEQ = 8192

NUM_CORES = 2
NUM_SUBCORES = 16
NW = NUM_CORES * NUM_SUBCORES
S_PER_W = SEQ // NW
K = 8
N_CHUNKS = S_PER_W // K
DEPTH = 4
PREF = 3
N_ROUNDS = N_CHUNKS // DEPTH
LANES = 16
VPR = D_MODEL // LANES

_mesh = plsc.VectorSubcoreMesh(core_axis_name="c", subcore_axis_name="s")


@functools.partial(
    pl.kernel,
    out_type=jax.ShapeDtypeStruct((BATCH, SEQ, D_MODEL), jnp.float32),
    mesh=_mesh,
    scratch_types=[
        pltpu.VMEM((S_PER_W,), jnp.int32),
        pltpu.VMEM((DEPTH, K, D_MODEL), jnp.float32),
        pltpu.VMEM((DEPTH, BATCH, K, D_MODEL), jnp.float32),
        pltpu.SemaphoreType.DMA((DEPTH,)),
        pltpu.SemaphoreType.DMA((DEPTH,)),
        pltpu.SemaphoreType.DMA((DEPTH,)),
    ],
)
def _pe_add(inp_hbm, idx_hbm, pe_hbm, out_hbm, idx_v, pe_buf, in_buf,
            sem_pe, sem_in, sem_out):
    wid = lax.axis_index("s") * NUM_CORES + lax.axis_index("c")
    base = wid * S_PER_W

    def issue_pe(c, slot):
        off = pl.multiple_of(c * K, 8)
        pltpu.async_copy(pe_hbm.at[idx_v.at[pl.ds(off, K)]],
                         pe_buf.at[slot], sem_pe.at[slot])

    def issue_in(c, slot):
        s0 = pl.multiple_of(base + c * K, 8)
        pltpu.async_copy(inp_hbm.at[:, pl.ds(s0, K)],
                         in_buf.at[slot], sem_in.at[slot])

    def issue_loads(c, slot):
        issue_pe(c, slot)
        issue_in(c, slot)

    def wait_loads(slot):
        pltpu.make_async_copy(pe_hbm.at[pl.ds(0, K)], pe_buf.at[slot],
                              sem_pe.at[slot]).wait()
        pltpu.make_async_copy(inp_hbm.at[:, pl.ds(0, K)],
                              in_buf.at[slot], sem_in.at[slot]).wait()

    def issue_stores(c, slot):
        s0 = pl.multiple_of(base + c * K, 8)
        pltpu.async_copy(in_buf.at[slot], out_hbm.at[:, pl.ds(s0, K)],
                         sem_out.at[slot])

    def wait_stores(slot):
        pltpu.make_async_copy(in_buf.at[slot],
                              out_hbm.at[:, pl.ds(0, K)],
                              sem_out.at[slot]).wait()

    for c in range(PREF):
        issue_in(c, c % DEPTH)
    pltpu.sync_copy(idx_hbm.at[pl.ds(pl.multiple_of(base, 8), S_PER_W)],
                    idx_v)
    for c in range(PREF):
        issue_pe(c, c % DEPTH)

    def round_body(g, _):
        c0 = g * DEPTH
        for d in range(DEPTH):
            c = c0 + d
            nslot = (d + PREF) % DEPTH

            @pl.when(c + PREF < N_CHUNKS)
            def _(c=c, nslot=nslot):
                issue_pe(c + PREF, nslot)

                @pl.when(c >= DEPTH - PREF)
                def _():
                    wait_stores(nslot)
                issue_in(c + PREF, nslot)

            wait_loads(d)

            def add_rows(r, _, d=d):
                for v in range(VPR):
                    sl = pl.ds(v * LANES, LANES)
                    pv = pe_buf[d, r, sl]
                    plsc.addupdate(in_buf.at[d, 0, r, sl], pv)
                    plsc.addupdate(in_buf.at[d, 1, r, sl], pv)
                return 0

            lax.fori_loop(0, K, add_rows, 0)
            issue_stores(c, d)
        return 0

    lax.fori_loop(0, N_ROUNDS, round_body, 0)

    for slot in range(DEPTH):
        wait_stores(slot)


def kernel(input, indices, pe):
    return _pe_add(input, indices.astype(jnp.int32), pe)
